# Initial kernel scaffold; baseline (speedup 1.0000x reference)
import jax
import jax.numpy as jnp
from jax.experimental import pallas as pl


def kernel(lyric, features, singer, genre, id, W_feat, b_feat, E_singer, E_genre, E_music, W_out, b_out):
    raise NotImplementedError("write your pallas kernel here")


# speedup vs baseline: 1.4793x; 1.4793x over previous
import functools

import jax
import jax.numpy as jnp
from jax import lax
from jax.experimental import pallas as pl
from jax.experimental.pallas import tpu as pltpu
from jax.experimental.pallas import tpu_sc as plsc

B = 16384
EMB = 64
OUT = 512
NC = 2
NS = 16
NW = NC * NS
BPW = B // NW
CHUNK = 128
NCHUNK = BPW // CHUNK


def _sc_gather_body(em_hbm, es_hbm, eg_hbm, idm_hbm, ids_hbm, idg_hbm,
                    om_hbm, os_hbm, og_hbm,
                    idx_v, rows_m, rows_s, rows_g, sem):
    wid = lax.axis_index("s") * NC + lax.axis_index("c")
    row0 = wid * NCHUNK

    pltpu.sync_copy(idm_hbm.at[pl.ds(row0, NCHUNK)], idx_v.at[0])
    pltpu.sync_copy(ids_hbm.at[pl.ds(row0, NCHUNK)], idx_v.at[1])
    pltpu.sync_copy(idg_hbm.at[pl.ds(row0, NCHUNK)], idx_v.at[2])

    copies = []
    for j in range(NCHUNK):
        dst = rows_m.at[pl.ds(j * CHUNK, CHUNK)]
        copies.append(pltpu.async_copy(em_hbm.at[idx_v.at[0, j]], dst, sem))
    for j in range(NCHUNK):
        dst = rows_s.at[pl.ds(j * CHUNK, CHUNK)]
        copies.append(pltpu.async_copy(es_hbm.at[idx_v.at[1, j]], dst, sem))
    for j in range(NCHUNK):
        dst = rows_g.at[pl.ds(j * CHUNK, CHUNK)]
        copies.append(pltpu.async_copy(eg_hbm.at[idx_v.at[2, j]], dst, sem))
    for c in copies:
        c.wait()

    base = wid * BPW
    pltpu.sync_copy(rows_m, om_hbm.at[pl.ds(base, BPW)])
    pltpu.sync_copy(rows_s, os_hbm.at[pl.ds(base, BPW)])
    pltpu.sync_copy(rows_g, og_hbm.at[pl.ds(base, BPW)])


@jax.jit
def _sc_gather(E_music, E_singer, E_genre, idm, ids, idg):
    mesh = plsc.VectorSubcoreMesh(core_axis_name="c", subcore_axis_name="s",
                                  num_cores=NC, num_subcores=NS)
    out_type = (
        jax.ShapeDtypeStruct((B, EMB), jnp.float32),
        jax.ShapeDtypeStruct((B, EMB), jnp.float32),
        jax.ShapeDtypeStruct((B, EMB), jnp.float32),
    )
    scratch = [
        pltpu.VMEM((3, NCHUNK, CHUNK), jnp.int32),
        pltpu.VMEM((BPW, EMB), jnp.float32),
        pltpu.VMEM((BPW, EMB), jnp.float32),
        pltpu.VMEM((BPW, EMB), jnp.float32),
        pltpu.SemaphoreType.DMA,
    ]
    k = pl.kernel(_sc_gather_body, out_type=out_type, mesh=mesh,
                  scratch_types=scratch,
                  compiler_params=pltpu.CompilerParams(
                      use_tc_tiling_on_sc=False))
    return k(E_music, E_singer, E_genre, idm, ids, idg)


def _mm_body(m_ref, s_ref, g_ref, w1_ref, w2_ref, w3_ref, b_ref, o_ref):
    acc = jnp.dot(m_ref[...], w1_ref[...], preferred_element_type=jnp.float32)
    acc += jnp.dot(s_ref[...], w2_ref[...], preferred_element_type=jnp.float32)
    acc += jnp.dot(g_ref[...], w3_ref[...], preferred_element_type=jnp.float32)
    o_ref[...] = acc + b_ref[...]


@functools.partial(jax.jit, static_argnames=("bb",))
def _tc_project(memb, sing, gen, w1, w2, w3, b, bb=1024):
    grid = (B // bb,)
    emb_spec = pl.BlockSpec((bb, EMB), lambda i: (i, 0))
    w_spec = pl.BlockSpec((EMB, OUT), lambda i: (0, 0))
    return pl.pallas_call(
        _mm_body,
        grid=grid,
        in_specs=[emb_spec, emb_spec, emb_spec, w_spec, w_spec, w_spec,
                  pl.BlockSpec((1, OUT), lambda i: (0, 0))],
        out_specs=pl.BlockSpec((bb, OUT), lambda i: (i, 0)),
        out_shape=jax.ShapeDtypeStruct((B, OUT), jnp.float32),
    )(memb, sing, gen, w1, w2, w3, b)


def kernel(lyric, features, singer, genre, id, W_feat, b_feat,
           E_singer, E_genre, E_music, W_out, b_out):
    idm = id.astype(jnp.int32).reshape(B // CHUNK, CHUNK)
    ids = singer.astype(jnp.int32).reshape(B // CHUNK, CHUNK)
    idg = genre.astype(jnp.int32).reshape(B // CHUNK, CHUNK)
    memb, sing, gen = _sc_gather(E_music, E_singer, E_genre, idm, ids, idg)
    WT = W_out.T
    return _tc_project(memb, sing, gen, WT[:EMB], WT[EMB:2 * EMB],
                       WT[2 * EMB:], b_out.reshape(1, OUT))


# speedup vs baseline: 2.4544x; 1.6591x over previous
import functools

import jax
import jax.numpy as jnp
from jax import lax
from jax.experimental import pallas as pl
from jax.experimental.pallas import tpu as pltpu
from jax.experimental.pallas import tpu_sc as plsc

B = 16384
EMB = 64
OUT = 512
N_SINGERS = 417
N_GENRES = 18
NC = 2
NS = 16
NW = NC * NS
BPW = B // NW
CHUNK = 128
NCHUNK = BPW // CHUNK


def _sc_gather_body(em_hbm, idm_hbm, om_hbm, idx_v, rows_v, sem):
    wid = lax.axis_index("s") * NC + lax.axis_index("c")
    row0 = wid * NCHUNK

    pltpu.sync_copy(idm_hbm.at[pl.ds(row0, NCHUNK)], idx_v)
    copies = []
    for j in range(NCHUNK):
        dst = rows_v.at[pl.ds(j * CHUNK, CHUNK)]
        copies.append(pltpu.async_copy(em_hbm.at[idx_v.at[j]], dst, sem))
    for c in copies:
        c.wait()
    pltpu.sync_copy(rows_v, om_hbm.at[pl.ds(wid * BPW, BPW)])


@jax.jit
def _sc_gather(E_music, idm):
    mesh = plsc.VectorSubcoreMesh(core_axis_name="c", subcore_axis_name="s",
                                  num_cores=NC, num_subcores=NS)
    k = pl.kernel(_sc_gather_body,
                  out_type=jax.ShapeDtypeStruct((B, EMB), jnp.float32),
                  mesh=mesh,
                  scratch_types=[
                      pltpu.VMEM((NCHUNK, CHUNK), jnp.int32),
                      pltpu.VMEM((BPW, EMB), jnp.float32),
                      pltpu.SemaphoreType.DMA,
                  ],
                  compiler_params=pltpu.CompilerParams(
                      use_tc_tiling_on_sc=False))
    return k(E_music, idm)


def _mm_body(m_ref, sidx_ref, gidx_ref, es_ref, eg_ref,
             w1_ref, w2_ref, w3_ref, b_ref, o_ref):
    bb = m_ref.shape[0]
    sidx = sidx_ref[0, 0, :]
    gidx = gidx_ref[0, 0, :]
    s_oh = (sidx[:, None] ==
            lax.broadcasted_iota(jnp.int32, (bb, N_SINGERS), 1)
            ).astype(jnp.float32)
    g_oh = (gidx[:, None] ==
            lax.broadcasted_iota(jnp.int32, (bb, N_GENRES), 1)
            ).astype(jnp.float32)
    s_emb = jnp.dot(s_oh, es_ref[...], preferred_element_type=jnp.float32)
    g_emb = jnp.dot(g_oh, eg_ref[...], preferred_element_type=jnp.float32)
    acc = jnp.dot(m_ref[...], w1_ref[...], preferred_element_type=jnp.float32)
    acc += jnp.dot(s_emb, w2_ref[...], preferred_element_type=jnp.float32)
    acc += jnp.dot(g_emb, w3_ref[...], preferred_element_type=jnp.float32)
    o_ref[...] = acc + b_ref[...]


@functools.partial(jax.jit, static_argnames=("bb",))
def _tc_project(memb, sidx, gidx, E_singer, E_genre, w1, w2, w3, b, bb=1024):
    grid = (B // bb,)
    idx_spec = pl.BlockSpec((1, 1, bb), lambda i: (i, 0, 0))
    w_spec = pl.BlockSpec((EMB, OUT), lambda i: (0, 0))
    return pl.pallas_call(
        _mm_body,
        grid=grid,
        in_specs=[
            pl.BlockSpec((bb, EMB), lambda i: (i, 0)),
            idx_spec, idx_spec,
            pl.BlockSpec((N_SINGERS, EMB), lambda i: (0, 0)),
            pl.BlockSpec((N_GENRES, EMB), lambda i: (0, 0)),
            w_spec, w_spec, w_spec,
            pl.BlockSpec((1, OUT), lambda i: (0, 0)),
        ],
        out_specs=pl.BlockSpec((bb, OUT), lambda i: (i, 0)),
        out_shape=jax.ShapeDtypeStruct((B, OUT), jnp.float32),
    )(memb, sidx, gidx, E_singer, E_genre, w1, w2, w3, b)


def kernel(lyric, features, singer, genre, id, W_feat, b_feat,
           E_singer, E_genre, E_music, W_out, b_out):
    bb = 1024
    idm = id.astype(jnp.int32).reshape(B // CHUNK, CHUNK)
    sidx = singer.astype(jnp.int32).reshape(B // bb, 1, bb)
    gidx = genre.astype(jnp.int32).reshape(B // bb, 1, bb)
    memb = _sc_gather(E_music, idm)
    WT = W_out.T
    return _tc_project(memb, sidx, gidx, E_singer, E_genre,
                       WT[:EMB], WT[EMB:2 * EMB], WT[2 * EMB:],
                       b_out.reshape(1, OUT), bb=bb)
